# baseline (speedup 1.0000x reference)
import functools

import jax
import jax.numpy as jnp
from jax import lax
from jax.experimental import pallas as pl
from jax.experimental.pallas import tpu as pltpu
from jax.experimental.pallas import tpu_sc as plsc

_N_EMBD = 768
_BATCH = 32
_SEQ = 1024
_NC, _NS = 2, 16
_NW = _NC * _NS
_K = _SEQ // _NW
_L = 16
_NBUF = 8
_HR = _K // 2
_NT = _BATCH * 2
_AHEAD = 4
_RUNROLL = 4


def _emb_body(data_hbm, wte_hbm, wpe_hbm, out_hbm,
              idx_all, pos_v, bufs, isem, gsems, osems):
    wid = lax.axis_index("s") * _NC + lax.axis_index("c")
    base_s = wid * _K

    pltpu.sync_copy(wpe_hbm.at[pl.ds(base_s, _K)], pos_v)

    for b in range(_BATCH):
        pltpu.async_copy(
            data_hbm.at[pl.ds(b * _SEQ + base_s, _K)], idx_all.at[b], isem)
    for b in range(_BATCH):
        pltpu.make_async_copy(
            data_hbm.at[pl.ds(b * _SEQ + base_s, _K)], idx_all.at[b], isem
        ).wait()

    def idx_of(t):
        return idx_all.at[t // 2, pl.ds((t % 2) * _HR, _HR)]

    def gather(t, h):
        pltpu.async_copy(wte_hbm.at[idx_of(t)], bufs[h], gsems[h])

    def gather_wait(t, h):
        pltpu.make_async_copy(wte_hbm.at[idx_of(t)], bufs[h], gsems[h]).wait()

    def out_ref(t):
        return out_hbm.at[t // 2, pl.ds(base_s + (t % 2) * _HR, _HR)]

    def out_wait(h):
        pltpu.make_async_copy(wte_hbm.at[pl.ds(0, _HR)], bufs[h], osems[h]).wait()

    for t in range(_AHEAD):
        gather(t, t % _NBUF)

    def step(i, _):
        for u in range(_NBUF):
            t = i * _NBUF + u
            q = (u + _AHEAD) % _NBUF

            @pl.when(t + _AHEAD < _NT)
            def _():
                @pl.when(t >= _AHEAD)
                def _():
                    out_wait(q)
                gather(t + _AHEAD, q)

            gather_wait(t, u)
            buf = bufs[u]
            roff = (t % 2) * _HR

            def add_rows(r0, _):
                for dr in range(_RUNROLL):
                    r = r0 * _RUNROLL + dr
                    for j in range(_N_EMBD // _L):
                        sl = pl.ds(j * _L, _L)
                        plsc.addupdate(buf.at[r, sl], pos_v[roff + r, sl])
                return 0

            lax.fori_loop(0, _HR // _RUNROLL, add_rows, 0)
            pltpu.async_copy(buf, out_ref(t), osems[u])
        return 0

    lax.fori_loop(0, _NT // _NBUF, step, 0)

    for h in range(_NBUF):
        out_wait(h)


@jax.jit
def kernel(data, wte, wpe):
    mesh = plsc.VectorSubcoreMesh(
        core_axis_name="c", subcore_axis_name="s",
        num_cores=_NC, num_subcores=_NS,
    )
    run = functools.partial(
        pl.kernel,
        out_type=jax.ShapeDtypeStruct((_BATCH, _SEQ, _N_EMBD), jnp.float32),
        mesh=mesh,
        scratch_types=[
            pltpu.VMEM((_BATCH, _K), jnp.int32),
            pltpu.VMEM((_K, _N_EMBD), jnp.float32),
            tuple(pltpu.VMEM((_HR, _N_EMBD), jnp.float32)
                  for _ in range(_NBUF)),
            pltpu.SemaphoreType.DMA,
            tuple(pltpu.SemaphoreType.DMA for _ in range(_NBUF)),
            tuple(pltpu.SemaphoreType.DMA for _ in range(_NBUF)),
        ],
    )(_emb_body)
    return run(data.reshape(-1), wte, wpe)


# speedup vs baseline: 1.5425x; 1.5425x over previous
import functools

import jax
import jax.numpy as jnp
from jax import lax
from jax.experimental import pallas as pl
from jax.experimental.pallas import tpu as pltpu
from jax.experimental.pallas import tpu_sc as plsc

_N_EMBD = 768
_BATCH = 32
_SEQ = 1024
_NC, _NS = 2, 16
_NW = _NC * _NS
_K = _SEQ // _NW
_L = 16
_NBUF = 4
_RUNROLL = 4


def _emb_body(data_hbm, wte_hbm, wpe_hbm, out_hbm,
              idx_all, pos_v, bufs, gsems, osems):
    wid = lax.axis_index("s") * _NC + lax.axis_index("c")
    base_s = wid * _K

    pltpu.sync_copy(wpe_hbm.at[pl.ds(base_s, _K)], pos_v)

    pltpu.sync_copy(
        data_hbm.at[pl.ds(wid * (_BATCH * _K), _BATCH * _K)], idx_all)

    def gather(b, p):
        pltpu.async_copy(
            wte_hbm.at[idx_all.at[pl.ds(b * _K, _K)]], bufs[p], gsems[p])

    def gather_wait(b, p):
        pltpu.make_async_copy(
            wte_hbm.at[idx_all.at[pl.ds(b * _K, _K)]], bufs[p], gsems[p]).wait()

    def out_wait(p):
        pltpu.make_async_copy(wte_hbm.at[pl.ds(0, _K)], bufs[p], osems[p]).wait()

    gather(0, 0)
    gather(1, 1)

    def step(i, _):
        for p in range(_NBUF):
            b = i * _NBUF + p
            q = (p + 2) % _NBUF

            @pl.when(b + 2 < _BATCH)
            def _():
                @pl.when(b >= 2)
                def _():
                    out_wait(q)
                gather(b + 2, q)

            gather_wait(b, p)
            buf = bufs[p]

            def add_rows(r0, _):
                for dr in range(_RUNROLL):
                    r = r0 * _RUNROLL + dr
                    for j in range(_N_EMBD // _L):
                        sl = pl.ds(j * _L, _L)
                        plsc.addupdate(buf.at[r, sl], pos_v[r, sl])
                return 0

            lax.fori_loop(0, _K // _RUNROLL, add_rows, 0)
            pltpu.async_copy(buf, out_hbm.at[b, pl.ds(base_s, _K)], osems[p])
        return 0

    lax.fori_loop(0, _BATCH // _NBUF, step, 0)

    for p in range(_NBUF):
        out_wait(p)


@jax.jit
def kernel(data, wte, wpe):
    mesh = plsc.VectorSubcoreMesh(
        core_axis_name="c", subcore_axis_name="s",
        num_cores=_NC, num_subcores=_NS,
    )
    run = functools.partial(
        pl.kernel,
        out_type=jax.ShapeDtypeStruct((_BATCH, _SEQ, _N_EMBD), jnp.float32),
        mesh=mesh,
        scratch_types=[
            pltpu.VMEM((_BATCH * _K,), jnp.int32),
            pltpu.VMEM((_K, _N_EMBD), jnp.float32),
            tuple(pltpu.VMEM((_K, _N_EMBD), jnp.float32)
                  for _ in range(_NBUF)),
            tuple(pltpu.SemaphoreType.DMA for _ in range(_NBUF)),
            tuple(pltpu.SemaphoreType.DMA for _ in range(_NBUF)),
        ],
    )(_emb_body)
    data_t = data.reshape(_BATCH, _NW, _K).transpose(1, 0, 2).reshape(-1)
    return run(data_t, wte, wpe)
